# baseline (speedup 1.0000x reference)
import math

import jax
import jax.numpy as jnp
import numpy as np
from jax.experimental import pallas as pl
from jax.experimental.pallas import tpu as pltpu

DEPTH = 5
N_LEAVES = 2 ** DEPTH
N_NODES = 2 ** DEPTH - 1
INPUT_WIDTH = 1024
LEAF_WIDTH = 64
OUTPUT_WIDTH = 1024
HIDDEN = N_LEAVES * LEAF_WIDTH

BATCH_TILE = 512


def _path_matrices():
    p_pos = np.zeros((N_LEAVES, N_LEAVES), np.float32)
    p_neg = np.zeros((N_LEAVES, N_LEAVES), np.float32)
    for l in range(N_LEAVES):
        for d in range(DEPTH):
            node = 2 ** d - 1 + (l >> (DEPTH - d))
            bit = (l >> (DEPTH - 1 - d)) & 1
            if bit:
                p_pos[node, l] = 1.0
            else:
                p_neg[node, l] = 1.0
    return p_pos, p_neg


_P_POS, _P_NEG = _path_matrices()
_EXPAND = np.kron(np.eye(N_LEAVES, dtype=np.float32), np.ones((1, LEAF_WIDTH), np.float32))


def _log_sigmoid(z):
    return -(jnp.maximum(-z, 0.0) + jnp.log(1.0 + jnp.exp(-jnp.abs(z))))


def _fff_body(x_ref, nwt_ref, nb_ref, w1_ref, b1_ref, w2_ref, b2_ref,
              ppos_ref, pneg_ref, exp_ref, o_ref, hm_ref, mix_ref):
    o_ref[...] = (jnp.dot(hm_ref[...], w2_ref[...],
                          preferred_element_type=jnp.float32)
                  + jnp.dot(mix_ref[...], b2_ref[...],
                            preferred_element_type=jnp.float32))

    x = x_ref[...]
    z = jnp.dot(x, nwt_ref[...], preferred_element_type=jnp.float32) + nb_ref[...]
    log_mix = (jnp.dot(_log_sigmoid(z), ppos_ref[...],
                       preferred_element_type=jnp.float32)
               + jnp.dot(_log_sigmoid(-z), pneg_ref[...],
                         preferred_element_type=jnp.float32))
    mix = jnp.exp(log_mix)
    h = jnp.maximum(
        jnp.dot(x, w1_ref[...], preferred_element_type=jnp.float32) + b1_ref[...],
        0.0)
    hm_ref[...] = h * jnp.dot(mix, exp_ref[...], preferred_element_type=jnp.float32)
    mix_ref[...] = mix


def kernel(x, node_weights, node_biases, w1s, b1s, w2s, b2s):
    batch = x.shape[0]
    x = x.reshape(batch, INPUT_WIDTH)
    nwt = jnp.concatenate(
        [node_weights, jnp.zeros((1, INPUT_WIDTH), node_weights.dtype)], axis=0).T
    nb = jnp.concatenate(
        [node_biases[:, 0], jnp.zeros((1,), node_biases.dtype)])[None, :]
    b1 = b1s.reshape(1, HIDDEN)
    w1 = jnp.transpose(w1s, (1, 0, 2)).reshape(INPUT_WIDTH, HIDDEN)
    w2 = w2s.reshape(HIDDEN, OUTPUT_WIDTH)

    n_tiles = batch // BATCH_TILE
    full = lambda shape: pl.BlockSpec(shape, lambda i: tuple(0 for _ in shape))
    out = pl.pallas_call(
        _fff_body,
        grid=(n_tiles + 1,),
        in_specs=[
            pl.BlockSpec((BATCH_TILE, INPUT_WIDTH),
                         lambda i: (jnp.minimum(i, n_tiles - 1), 0)),
            full((INPUT_WIDTH, N_LEAVES)),
            full((1, N_LEAVES)),
            full((INPUT_WIDTH, HIDDEN)),
            full((1, HIDDEN)),
            full((HIDDEN, OUTPUT_WIDTH)),
            full((N_LEAVES, OUTPUT_WIDTH)),
            full((N_LEAVES, N_LEAVES)),
            full((N_LEAVES, N_LEAVES)),
            full((N_LEAVES, HIDDEN)),
        ],
        out_specs=pl.BlockSpec((BATCH_TILE, OUTPUT_WIDTH),
                               lambda i: (jnp.maximum(i - 1, 0), 0)),
        out_shape=jax.ShapeDtypeStruct((batch, OUTPUT_WIDTH), jnp.float32),
        scratch_shapes=[
            pltpu.VMEM((BATCH_TILE, HIDDEN), jnp.float32),
            pltpu.VMEM((BATCH_TILE, N_LEAVES), jnp.float32),
        ],
        compiler_params=pltpu.CompilerParams(
            dimension_semantics=("arbitrary",),
        ),
    )(x, nwt, nb, w1, b1, w2, b2s,
      jnp.asarray(_P_POS), jnp.asarray(_P_NEG), jnp.asarray(_EXPAND))
    return out


# speedup vs baseline: 1.1477x; 1.1477x over previous
import math

import jax
import jax.numpy as jnp
import numpy as np
from jax.experimental import pallas as pl
from jax.experimental.pallas import tpu as pltpu

DEPTH = 5
N_LEAVES = 2 ** DEPTH
N_NODES = 2 ** DEPTH - 1
INPUT_WIDTH = 1024
LEAF_WIDTH = 64
OUTPUT_WIDTH = 1024
HIDDEN = N_LEAVES * LEAF_WIDTH

BATCH_TILE = 1024


def _path_matrices():
    p_pos = np.zeros((N_NODES, N_LEAVES), np.float32)
    p_neg = np.zeros((N_NODES, N_LEAVES), np.float32)
    for l in range(N_LEAVES):
        for d in range(DEPTH):
            node = 2 ** d - 1 + (l >> (DEPTH - d))
            bit = (l >> (DEPTH - 1 - d)) & 1
            if bit:
                p_pos[node, l] = 1.0
            else:
                p_neg[node, l] = 1.0
    return p_pos, p_neg


_P_POS, _P_NEG = _path_matrices()
_EXPAND = np.kron(np.eye(N_LEAVES, dtype=np.float32), np.ones((1, LEAF_WIDTH), np.float32))

_NT = (((1,), (1,)), ((), ()))


def _log_sigmoid(z):
    return -(jnp.maximum(-z, 0.0) + jnp.log(1.0 + jnp.exp(-jnp.abs(z))))


def _fff_body(x_ref, nw_ref, nb_ref, w1_ref, b1_ref, w2_ref, b2_ref,
              ppos_ref, pneg_ref, exp_ref, o_ref):
    x = x_ref[...]
    z = jax.lax.dot_general(x, nw_ref[...], _NT,
                            preferred_element_type=jnp.float32) + nb_ref[...]
    log_mix = (jnp.dot(_log_sigmoid(z), ppos_ref[...],
                       preferred_element_type=jnp.float32)
               + jnp.dot(_log_sigmoid(-z), pneg_ref[...],
                         preferred_element_type=jnp.float32))
    mix = jnp.exp(log_mix)
    h = jnp.maximum(
        jnp.dot(x.astype(jnp.bfloat16), w1_ref[...],
                preferred_element_type=jnp.float32) + b1_ref[...],
        0.0)
    hm = h * jnp.dot(mix, exp_ref[...], preferred_element_type=jnp.float32)
    o_ref[...] = (jnp.dot(hm, w2_ref[...], preferred_element_type=jnp.float32)
                  + jnp.dot(mix, b2_ref[...], preferred_element_type=jnp.float32))


def kernel(x, node_weights, node_biases, w1s, b1s, w2s, b2s):
    batch = x.shape[0]
    x = x.reshape(batch, INPUT_WIDTH)
    nb = node_biases.reshape(1, N_NODES)
    b1 = b1s.reshape(1, HIDDEN)
    w1 = jnp.transpose(w1s, (1, 0, 2)).reshape(
        INPUT_WIDTH, HIDDEN).astype(jnp.bfloat16)
    w2 = w2s.reshape(HIDDEN, OUTPUT_WIDTH)

    n_tiles = batch // BATCH_TILE
    full = lambda shape: pl.BlockSpec(shape, lambda i: tuple(0 for _ in shape))
    out = pl.pallas_call(
        _fff_body,
        grid=(n_tiles,),
        in_specs=[
            pl.BlockSpec((BATCH_TILE, INPUT_WIDTH), lambda i: (i, 0)),
            full((N_NODES, INPUT_WIDTH)),
            full((1, N_NODES)),
            full((INPUT_WIDTH, HIDDEN)),
            full((1, HIDDEN)),
            full((HIDDEN, OUTPUT_WIDTH)),
            full((N_LEAVES, OUTPUT_WIDTH)),
            full((N_NODES, N_LEAVES)),
            full((N_NODES, N_LEAVES)),
            full((N_LEAVES, HIDDEN)),
        ],
        out_specs=pl.BlockSpec((BATCH_TILE, OUTPUT_WIDTH), lambda i: (i, 0)),
        out_shape=jax.ShapeDtypeStruct((batch, OUTPUT_WIDTH), jnp.float32),
        compiler_params=pltpu.CompilerParams(
            dimension_semantics=("arbitrary",),
        ),
    )(x, node_weights, nb, w1, b1, w2, b2s,
      jnp.asarray(_P_POS), jnp.asarray(_P_NEG), jnp.asarray(_EXPAND))
    return out
